# baseline (speedup 1.0000x reference)
import functools

import jax
import jax.numpy as jnp
from jax import lax
from jax.experimental import pallas as pl
from jax.experimental.pallas import tpu as pltpu
from jax.experimental.pallas import tpu_sc as plsc

_NUM_CORES = 2
_NUM_SUBCORES = 16
_NUM_WORKERS = _NUM_CORES * _NUM_SUBCORES
_CHUNK = 32
_NBUF = 2
_SC_TOKENS = 2048
_TBLK = 2048


@functools.lru_cache(maxsize=None)
def _make_sc_lookup(sc_tokens, num_tokens, hidden):
    b_per_w = sc_tokens // _NUM_WORKERS
    n_chunks = b_per_w // _CHUNK
    assert n_chunks % _NBUF == 0
    mesh = plsc.VectorSubcoreMesh(core_axis_name="c", subcore_axis_name="s")

    @functools.partial(
        pl.kernel,
        out_type=jax.ShapeDtypeStruct((num_tokens, hidden), jnp.float32),
        mesh=mesh,
        scratch_types=[
            pltpu.VMEM((n_chunks, _CHUNK), jnp.int32),
            pltpu.VMEM((_NBUF, _CHUNK, hidden), jnp.float32),
        ] + [pltpu.SemaphoreType.DMA] * (2 * _NBUF),
    )
    def lookup(ids_hbm, table_hbm, out_hbm, idx_v, rows_v, *sems):
        gsems = sems[:_NBUF]
        ssems = sems[_NBUF:]
        wid = lax.axis_index("s") * _NUM_CORES + lax.axis_index("c")
        base = wid * b_per_w

        pltpu.sync_copy(ids_hbm.at[wid], idx_v)

        def gather_start(j, b):
            pltpu.async_copy(
                table_hbm.at[idx_v.at[j]], rows_v.at[b], gsems[b])

        def gather_wait(j, b):
            pltpu.make_async_copy(
                table_hbm.at[idx_v.at[j]], rows_v.at[b], gsems[b]).wait()

        for b in range(_NBUF):
            gather_start(b, b)

        @pl.loop(0, n_chunks - _NBUF, step=_NBUF)
        def _(g):
            for b in range(_NBUF):
                j = g + b
                gather_wait(j, b)
                scat = pltpu.async_copy(
                    rows_v.at[b],
                    out_hbm.at[pl.ds(base + j * _CHUNK, _CHUNK)],
                    ssems[b])
                scat.wait()
                gather_start(j + _NBUF, b)

        for b in range(_NBUF):
            j = n_chunks - _NBUF + b
            gather_wait(j, b)
            pltpu.sync_copy(
                rows_v.at[b],
                out_hbm.at[pl.ds(base + j * _CHUNK, _CHUNK)])

    return lookup


@functools.lru_cache(maxsize=None)
def _make_tc_lookup(sc_tokens, num_tokens, hidden):
    assert sc_tokens % _TBLK == 0
    grid = (num_tokens - sc_tokens) // _TBLK
    blk0 = sc_tokens // _TBLK

    def body(ids_ref, table_ref, prev_ref, out_ref):
        del prev_ref
        ids = ids_ref[...]
        t0 = table_ref[0:1, :]
        t1 = table_ref[1:2, :]
        m = jnp.broadcast_to(ids == 0, (_TBLK, hidden))
        out_ref[...] = jnp.where(
            m,
            jnp.broadcast_to(t0, (_TBLK, hidden)),
            jnp.broadcast_to(t1, (_TBLK, hidden)))

    return pl.pallas_call(
        body,
        grid=(grid,),
        in_specs=[
            pl.BlockSpec((_TBLK, 1), lambda i: (i, 0)),
            pl.BlockSpec((2, hidden), lambda i: (0, 0)),
            pl.BlockSpec(memory_space=pltpu.MemorySpace.HBM),
        ],
        out_specs=pl.BlockSpec((_TBLK, hidden), lambda i: (blk0 + i, 0)),
        out_shape=jax.ShapeDtypeStruct((num_tokens, hidden), jnp.float32),
        input_output_aliases={2: 0},
    )


@jax.jit
def kernel(token_type_ids, table):
    batch, seq = token_type_ids.shape
    num_tokens = batch * seq
    vocab, hidden = table.shape
    tablef = table.astype(jnp.float32)
    ids_flat = token_type_ids.astype(jnp.int32).reshape(num_tokens)

    n_sc = _SC_TOKENS
    b_per_w = n_sc // _NUM_WORKERS
    n_chunks = b_per_w // _CHUNK
    ids_sc = ids_flat[:n_sc].reshape(_NUM_WORKERS, n_chunks, _CHUNK)
    table_rep = jnp.tile(tablef, (_NUM_WORKERS, 1))
    ids_sc = ids_sc + (vocab * jnp.arange(_NUM_WORKERS, dtype=jnp.int32)
                       )[:, None, None]
    out_sc = _make_sc_lookup(n_sc, num_tokens, hidden)(ids_sc, table_rep)

    n_tc = num_tokens - n_sc
    ids_tc = ids_flat[n_sc:].reshape(n_tc, 1)
    out = _make_tc_lookup(n_sc, num_tokens, hidden)(ids_tc, tablef, out_sc)
    return out.reshape(batch, seq, hidden)


# speedup vs baseline: 1.4825x; 1.4825x over previous
import functools

import jax
import jax.numpy as jnp
from jax.experimental import pallas as pl
from jax.experimental.pallas import tpu as pltpu

_TBLK = 2048


@functools.lru_cache(maxsize=None)
def _make_tc(num_tokens, hidden):
    grid = num_tokens // _TBLK

    def body(ids_ref, table_ref, out_ref):
        ids = ids_ref[...]
        t0 = table_ref[0:1, :]
        t1 = table_ref[1:2, :]
        m = jnp.broadcast_to(ids == 0, (_TBLK, hidden))
        out_ref[...] = jnp.where(
            m,
            jnp.broadcast_to(t0, (_TBLK, hidden)),
            jnp.broadcast_to(t1, (_TBLK, hidden)))

    return pl.pallas_call(
        body,
        grid=(grid,),
        in_specs=[
            pl.BlockSpec((_TBLK, 1), lambda i: (i, 0)),
            pl.BlockSpec((2, hidden), lambda i: (0, 0)),
        ],
        out_specs=pl.BlockSpec((_TBLK, hidden), lambda i: (i, 0)),
        out_shape=jax.ShapeDtypeStruct((num_tokens, hidden), jnp.float32),
    )


@jax.jit
def kernel(token_type_ids, table):
    batch, seq = token_type_ids.shape
    num_tokens = batch * seq
    hidden = table.shape[1]
    ids = token_type_ids.astype(jnp.int32).reshape(num_tokens, 1)
    out = _make_tc(num_tokens, hidden)(ids, table.astype(jnp.float32))
    return out.reshape(batch, seq, hidden)


# speedup vs baseline: 1.4973x; 1.0099x over previous
import functools

import jax
import jax.numpy as jnp
from jax.experimental import pallas as pl
from jax.experimental.pallas import tpu as pltpu

_TBLK = 4096


@functools.lru_cache(maxsize=None)
def _make_tc(num_tokens, hidden):
    grid = num_tokens // _TBLK

    def body(ids_ref, table_ref, out_ref):
        ids = ids_ref[...]
        t0 = table_ref[0:1, :]
        t1 = table_ref[1:2, :]
        m = jnp.broadcast_to(ids == 0, (_TBLK, hidden))
        out_ref[...] = jnp.where(
            m,
            jnp.broadcast_to(t0, (_TBLK, hidden)),
            jnp.broadcast_to(t1, (_TBLK, hidden)))

    return pl.pallas_call(
        body,
        grid=(grid,),
        in_specs=[
            pl.BlockSpec((_TBLK, 1), lambda i: (i, 0)),
            pl.BlockSpec((2, hidden), lambda i: (0, 0)),
        ],
        out_specs=pl.BlockSpec((_TBLK, hidden), lambda i: (i, 0)),
        out_shape=jax.ShapeDtypeStruct((num_tokens, hidden), jnp.float32),
    )


@jax.jit
def kernel(token_type_ids, table):
    batch, seq = token_type_ids.shape
    num_tokens = batch * seq
    hidden = table.shape[1]
    ids = token_type_ids.astype(jnp.int32).reshape(num_tokens, 1)
    out = _make_tc(num_tokens, hidden)(ids, table.astype(jnp.float32))
    return out.reshape(batch, seq, hidden)
